# baseline (speedup 1.0000x reference)
import functools

import jax
import jax.numpy as jnp
import numpy as np
from jax import lax
from jax.experimental import pallas as pl
from jax.experimental.pallas import tpu as pltpu
from jax.experimental.pallas import tpu_sc as plsc

N_NODES = 10000
N_EDGES = 320000
D = 128

NC = 2
NS = 16
NW = NC * NS
CHUNK = 128
GRP = 10
NGRP = 8
NCHUNK = GRP * NGRP
EPW = NCHUNK * CHUNK
N_PAD = 10112
ROWS_PER_TILE = N_PAD // NS

_PERM = np.empty((D,), dtype=np.int32)
for _g in range(D // 32):
    for _k in range(16):
        _PERM[32 * _g + 2 * _k] = 32 * _g + _k
        _PERM[32 * _g + 2 * _k + 1] = 32 * _g + 16 + _k

_mesh = plsc.VectorSubcoreMesh(core_axis_name="c", subcore_axis_name="s")


@functools.partial(
    pl.kernel,
    out_type=(
        jax.ShapeDtypeStruct((NC, N_PAD, D), jnp.float32),
        jax.ShapeDtypeStruct((NC, N_PAD, 16), jnp.float32),
    ),
    mesh=_mesh,
    compiler_params=pltpu.CompilerParams(use_tc_tiling_on_sc=False,
                                         needs_layout_passes=False),
    scratch_types=[
        pltpu.VMEM((GRP, CHUNK), jnp.int32),
        pltpu.VMEM((GRP, CHUNK), jnp.int32),
        pltpu.VMEM((CHUNK, D), jnp.bfloat16),
        pltpu.VMEM((CHUNK, D), jnp.bfloat16),
        pltpu.VMEM((CHUNK, D), jnp.float32),
        pltpu.VMEM((CHUNK, 16), jnp.float32),
        pltpu.VMEM((CHUNK, 16), jnp.float32),
        pltpu.VMEM_SHARED((N_PAD, D), jnp.float32),
        pltpu.VMEM_SHARED((N_PAD, 16), jnp.float32),
        pltpu.SemaphoreType.DMA,
        pltpu.SemaphoreType.DMA,
        pltpu.SemaphoreType.DMA,
        pltpu.SemaphoreType.DMA,
    ],
)
def _sc_aggregate(xh_hbm, src_hbm, dst_hbm, agg_out, deg_out,
                  src_v, dst_v, rb0_v, rb1_v, rf_v, ones_v, z16_v,
                  agg_sh, deg_sh, gsem0, gsem1, asem, dsem):
    c = lax.axis_index("c")
    s = lax.axis_index("s")
    wid = c * NS + s

    zero16 = jnp.zeros((16,), jnp.float32)
    one16 = jnp.ones((16,), jnp.float32)

    @pl.loop(0, CHUNK)
    def _fill_rows(i):
        for j in range(D // 16):
            rf_v[i, pl.ds(j * 16, 16)] = zero16
        ones_v[i, pl.ds(0, 16)] = one16
        z16_v[i, pl.ds(0, 16)] = zero16

    base = s * ROWS_PER_TILE
    nfull = ROWS_PER_TILE // CHUNK
    for t in range(nfull):
        pltpu.sync_copy(rf_v, agg_sh.at[pl.ds(base + t * CHUNK, CHUNK)])
        pltpu.sync_copy(z16_v, deg_sh.at[pl.ds(base + t * CHUNK, CHUNK)])
    rem = ROWS_PER_TILE % CHUNK
    if rem:
        pltpu.sync_copy(rf_v.at[pl.ds(0, rem)],
                        agg_sh.at[pl.ds(base + nfull * CHUNK, rem)])
        pltpu.sync_copy(z16_v.at[pl.ds(0, rem)],
                        deg_sh.at[pl.ds(base + nfull * CHUNK, rem)])
    plsc.subcore_barrier()

    bufs = (rb0_v, rb1_v)
    gsems = (gsem0, gsem1)

    @pl.loop(0, NGRP)
    def _edges(g):
        pltpu.sync_copy(src_hbm.at[wid, pl.ds(g * GRP, GRP)], src_v)
        pltpu.sync_copy(dst_hbm.at[wid, pl.ds(g * GRP, GRP)], dst_v)
        gd = [None, None]
        sd = None
        dd = None
        gd[0] = pltpu.async_copy(xh_hbm.at[src_v.at[0]], bufs[0], gsems[0])
        gd[1] = pltpu.async_copy(xh_hbm.at[src_v.at[1]], bufs[1], gsems[1])
        for j in range(GRP):
            b = j & 1
            gd[b].wait()
            if sd is not None:
                sd.wait()

            rb = bufs[b]

            @pl.loop(0, CHUNK, unroll=8)
            def _unpack(i):
                for q in range(D // 32):
                    w = plsc.bitcast(rb[i, pl.ds(32 * q, 32)], jnp.int32)
                    lo = plsc.bitcast(w << 16, jnp.float32)
                    hi = plsc.bitcast(w & jnp.int32(-65536), jnp.float32)
                    rf_v[i, pl.ds(32 * q, 16)] = lo
                    rf_v[i, pl.ds(32 * q + 16, 16)] = hi

            sd = pltpu.async_copy(rf_v, agg_sh.at[dst_v.at[j]], asem, add=True)
            if dd is not None:
                dd.wait()
            dd = pltpu.async_copy(ones_v, deg_sh.at[dst_v.at[j]], dsem, add=True)
            if j + 2 < GRP:
                gd[b] = pltpu.async_copy(xh_hbm.at[src_v.at[j + 2]],
                                         bufs[b], gsems[b])
        sd.wait()
        dd.wait()

    plsc.subcore_barrier()

    pltpu.sync_copy(agg_sh.at[pl.ds(base, ROWS_PER_TILE)],
                    agg_out.at[c, pl.ds(base, ROWS_PER_TILE)])
    pltpu.sync_copy(deg_sh.at[pl.ds(base, ROWS_PER_TILE)],
                    deg_out.at[c, pl.ds(base, ROWS_PER_TILE)])


_BLK = 2000


def _tc_body(x_ref, agg_ref, deg_ref, ws_ref, wn_ref, b_ref, o_ref):
    agg = agg_ref[0] + agg_ref[1]
    deg = deg_ref[0, :, 0:1] + deg_ref[1, :, 0:1]
    hn = agg / jnp.maximum(deg, 1.0)
    h = jnp.dot(x_ref[...], ws_ref[...], preferred_element_type=jnp.float32)
    h = h + jnp.dot(hn, wn_ref[...], preferred_element_type=jnp.float32)
    h = h + b_ref[...]
    h = jnp.where(h >= 0.0, h, h * 0.01)
    n2 = jnp.sum(h * h, axis=1, keepdims=True)
    o_ref[...] = h * lax.rsqrt(jnp.maximum(n2, 1e-24))


def _tc_finish(x, agg, deg, W_self, W_neigh, b2):
    grid = (N_NODES // _BLK,)
    return pl.pallas_call(
        _tc_body,
        grid=grid,
        in_specs=[
            pl.BlockSpec((_BLK, D), lambda i: (i, 0)),
            pl.BlockSpec((NC, _BLK, D), lambda i: (0, i, 0)),
            pl.BlockSpec((NC, _BLK, 16), lambda i: (0, i, 0)),
            pl.BlockSpec((D, D), lambda i: (0, 0)),
            pl.BlockSpec((D, D), lambda i: (0, 0)),
            pl.BlockSpec((1, D), lambda i: (0, 0)),
        ],
        out_specs=pl.BlockSpec((_BLK, D), lambda i: (i, 0)),
        out_shape=jax.ShapeDtypeStruct((N_NODES, D), jnp.float32),
    )(x, agg, deg, W_self, W_neigh, b2)


def kernel(x, edge_index, W_self, W_neigh, b):
    src = edge_index[0]
    dst = edge_index[1]
    xh = x[:, jnp.asarray(_PERM)].astype(jnp.bfloat16)
    real_pw = N_EDGES // NW
    pad_pw = EPW - real_pw
    pad_src = jnp.zeros((NW, pad_pw), jnp.int32)
    pad_dst = jnp.broadcast_to(
        N_NODES + (jnp.arange(pad_pw, dtype=jnp.int32) % (N_PAD - N_NODES)),
        (NW, pad_pw))
    src_p = jnp.concatenate([src.reshape(NW, real_pw), pad_src], axis=1)
    dst_p = jnp.concatenate([dst.reshape(NW, real_pw), pad_dst], axis=1)
    src_p = src_p.reshape(NW, NCHUNK, CHUNK)
    dst_p = dst_p.reshape(NW, NCHUNK, CHUNK)
    agg, deg = _sc_aggregate(xh, src_p, dst_p)
    return _tc_finish(x, agg, deg, W_self, W_neigh, b.reshape(1, D))


# speedup vs baseline: 1.0850x; 1.0850x over previous
import functools

import jax
import jax.numpy as jnp
import numpy as np
from jax import lax
from jax.experimental import pallas as pl
from jax.experimental.pallas import tpu as pltpu
from jax.experimental.pallas import tpu_sc as plsc

N_NODES = 10000
N_EDGES = 320000
D = 128

NC = 2
NS = 16
NW = NC * NS
CHUNK = 128
GRP = 8
NGRP = 10
NCHUNK = GRP * NGRP
EPW = NCHUNK * CHUNK
N_PAD = 10112
ROWS_PER_TILE = N_PAD // NS
DEGW = 8
HCH = CHUNK // 2

_PERM = np.empty((D,), dtype=np.int32)
for _g in range(D // 32):
    for _k in range(16):
        _PERM[32 * _g + 2 * _k] = 32 * _g + _k
        _PERM[32 * _g + 2 * _k + 1] = 32 * _g + 16 + _k

_DEGC = np.zeros((HCH + CHUNK, DEGW), dtype=np.float32)
_DEGC[:HCH, 0] = 1.0

_mesh = plsc.VectorSubcoreMesh(core_axis_name="c", subcore_axis_name="s")


@functools.partial(
    pl.kernel,
    out_type=(
        jax.ShapeDtypeStruct((NC, N_PAD, D), jnp.float32),
        jax.ShapeDtypeStruct((NC, N_PAD, DEGW), jnp.float32),
    ),
    mesh=_mesh,
    compiler_params=pltpu.CompilerParams(use_tc_tiling_on_sc=False,
                                         needs_layout_passes=False),
    scratch_types=[
        pltpu.VMEM((GRP, CHUNK), jnp.int32),
        pltpu.VMEM((2 * GRP, HCH), jnp.int32),
        pltpu.VMEM((CHUNK, D), jnp.bfloat16),
        pltpu.VMEM((CHUNK, D), jnp.bfloat16),
        pltpu.VMEM((CHUNK, D), jnp.float32),
        pltpu.VMEM((HCH, DEGW), jnp.float32),
        pltpu.VMEM((CHUNK, DEGW), jnp.float32),
        pltpu.VMEM_SHARED((N_PAD, D), jnp.float32),
        pltpu.VMEM_SHARED((N_PAD, DEGW), jnp.float32),
        pltpu.SemaphoreType.DMA,
        pltpu.SemaphoreType.DMA,
        pltpu.SemaphoreType.DMA,
        pltpu.SemaphoreType.DMA,
        pltpu.SemaphoreType.DMA,
        pltpu.SemaphoreType.DMA,
    ],
)
def _sc_aggregate(xh_hbm, src_hbm, dst_hbm, degc_hbm, agg_out, deg_out,
                  src_v, dst_v, rb0_v, rb1_v, rf_v, ones_v, z8_v,
                  agg_sh, deg_sh, gsem0, gsem1, asem0, asem1, dsem0, dsem1):
    c = lax.axis_index("c")
    s = lax.axis_index("s")
    wid = c * NS + s

    zero16 = jnp.zeros((16,), jnp.float32)

    pltpu.sync_copy(degc_hbm.at[pl.ds(0, HCH)], ones_v)
    pltpu.sync_copy(degc_hbm.at[pl.ds(HCH, CHUNK)], z8_v)

    @pl.loop(0, CHUNK, unroll=4)
    def _fill_rows(i):
        for j in range(D // 16):
            rf_v[i, pl.ds(j * 16, 16)] = zero16

    base = s * ROWS_PER_TILE
    nfull = ROWS_PER_TILE // CHUNK
    for t in range(nfull):
        pltpu.sync_copy(rf_v, agg_sh.at[pl.ds(base + t * CHUNK, CHUNK)])
        pltpu.sync_copy(z8_v, deg_sh.at[pl.ds(base + t * CHUNK, CHUNK)])
    rem = ROWS_PER_TILE % CHUNK
    if rem:
        pltpu.sync_copy(rf_v.at[pl.ds(0, rem)],
                        agg_sh.at[pl.ds(base + nfull * CHUNK, rem)])
        pltpu.sync_copy(z8_v.at[pl.ds(0, rem)],
                        deg_sh.at[pl.ds(base + nfull * CHUNK, rem)])
    plsc.subcore_barrier()

    rbufs = (rb0_v, rb1_v)
    gsems = (gsem0, gsem1)
    asems = (asem0, asem1)
    dsems = (dsem0, dsem1)

    @pl.loop(0, NGRP)
    def _edges(g):
        pltpu.sync_copy(src_hbm.at[wid, pl.ds(g * GRP, GRP)], src_v)
        pltpu.sync_copy(dst_hbm.at[wid, pl.ds(2 * g * GRP, 2 * GRP)], dst_v)
        gd = [None, None]
        sd = [None, None]
        dd = [None, None]
        gd[0] = pltpu.async_copy(xh_hbm.at[src_v.at[0]], rbufs[0], gsems[0])
        gd[1] = pltpu.async_copy(xh_hbm.at[src_v.at[1]], rbufs[1], gsems[1])
        for j in range(GRP):
            b = j & 1
            gd[b].wait()
            rb = rbufs[b]
            for h in range(2):
                if sd[h] is not None:
                    sd[h].wait()

                @pl.loop(HCH * h, HCH * h + HCH, unroll=8)
                def _widen(i):
                    for q in range(D // 32):
                        w = plsc.bitcast(rb[i, pl.ds(32 * q, 32)], jnp.int32)
                        rf_v[i, pl.ds(32 * q, 16)] = plsc.bitcast(
                            w << 16, jnp.float32)
                        rf_v[i, pl.ds(32 * q + 16, 16)] = plsc.bitcast(
                            w & jnp.int32(-65536), jnp.float32)

                k = 2 * j + h
                sd[h] = pltpu.async_copy(rf_v.at[pl.ds(HCH * h, HCH)],
                                         agg_sh.at[dst_v.at[k]],
                                         asems[h], add=True)
                if dd[h] is not None:
                    dd[h].wait()
                dd[h] = pltpu.async_copy(ones_v, deg_sh.at[dst_v.at[k]],
                                         dsems[h], add=True)
            if j + 2 < GRP:
                gd[b] = pltpu.async_copy(xh_hbm.at[src_v.at[j + 2]],
                                         rbufs[b], gsems[b])
        for h in range(2):
            sd[h].wait()
            dd[h].wait()

    plsc.subcore_barrier()

    pltpu.sync_copy(agg_sh.at[pl.ds(base, ROWS_PER_TILE)],
                    agg_out.at[c, pl.ds(base, ROWS_PER_TILE)])
    pltpu.sync_copy(deg_sh.at[pl.ds(base, ROWS_PER_TILE)],
                    deg_out.at[c, pl.ds(base, ROWS_PER_TILE)])


_BLK = 2000


def _tc_body(x_ref, agg_ref, deg_ref, ws_ref, wn_ref, b_ref, o_ref):
    agg = agg_ref[0] + agg_ref[1]
    deg = deg_ref[0, :, 0:1] + deg_ref[1, :, 0:1]
    hn = agg / jnp.maximum(deg, 1.0)
    h = jnp.dot(x_ref[...], ws_ref[...], preferred_element_type=jnp.float32)
    h = h + jnp.dot(hn, wn_ref[...], preferred_element_type=jnp.float32)
    h = h + b_ref[...]
    h = jnp.where(h >= 0.0, h, h * 0.01)
    n2 = jnp.sum(h * h, axis=1, keepdims=True)
    o_ref[...] = h * lax.rsqrt(jnp.maximum(n2, 1e-24))


def _tc_finish(x, agg, deg, W_self, W_neigh, b2):
    grid = (N_NODES // _BLK,)
    return pl.pallas_call(
        _tc_body,
        grid=grid,
        in_specs=[
            pl.BlockSpec((_BLK, D), lambda i: (i, 0)),
            pl.BlockSpec((NC, _BLK, D), lambda i: (0, i, 0)),
            pl.BlockSpec((NC, _BLK, DEGW), lambda i: (0, i, 0)),
            pl.BlockSpec((D, D), lambda i: (0, 0)),
            pl.BlockSpec((D, D), lambda i: (0, 0)),
            pl.BlockSpec((1, D), lambda i: (0, 0)),
        ],
        out_specs=pl.BlockSpec((_BLK, D), lambda i: (i, 0)),
        out_shape=jax.ShapeDtypeStruct((N_NODES, D), jnp.float32),
    )(x, agg, deg, W_self, W_neigh, b2)


def kernel(x, edge_index, W_self, W_neigh, b):
    src = edge_index[0]
    dst = edge_index[1]
    xh = x[:, jnp.asarray(_PERM)].astype(jnp.bfloat16)
    real_pw = N_EDGES // NW
    pad_pw = EPW - real_pw
    pad_src = jnp.zeros((NW, pad_pw), jnp.int32)
    pad_dst = jnp.broadcast_to(
        N_NODES + (jnp.arange(pad_pw, dtype=jnp.int32) % (N_PAD - N_NODES)),
        (NW, pad_pw))
    src_p = jnp.concatenate([src.reshape(NW, real_pw), pad_src], axis=1)
    dst_p = jnp.concatenate([dst.reshape(NW, real_pw), pad_dst], axis=1)
    src_p = src_p.reshape(NW, NCHUNK, CHUNK)
    dst_p = dst_p.reshape(NW, 2 * NCHUNK, HCH)
    agg, deg = _sc_aggregate(xh, src_p, dst_p, jnp.asarray(_DEGC))
    return _tc_finish(x, agg, deg, W_self, W_neigh, b.reshape(1, D))


# speedup vs baseline: 1.1084x; 1.0216x over previous
import functools

import jax
import jax.numpy as jnp
import numpy as np
from jax import lax
from jax.experimental import pallas as pl
from jax.experimental.pallas import tpu as pltpu
from jax.experimental.pallas import tpu_sc as plsc

N_NODES = 10000
N_EDGES = 320000
D = 128

NC = 2
NS = 16
NW = NC * NS
CHUNK = 128
GRP = 8
NGRP = 10
NCHUNK = GRP * NGRP
EPW = NCHUNK * CHUNK
N_PAD = 10112
ROWS_PER_TILE = N_PAD // NS
DEGW = 8
HCH = CHUNK // 2

_PERM = np.empty((D,), dtype=np.int32)
for _g in range(D // 32):
    for _k in range(16):
        _PERM[32 * _g + _k] = 32 * _g + 2 * _k
        _PERM[32 * _g + 16 + _k] = 32 * _g + 2 * _k + 1

_DEGC = np.zeros((HCH + CHUNK, DEGW), dtype=np.float32)
_DEGC[:HCH, 0] = 1.0

_mesh = plsc.VectorSubcoreMesh(core_axis_name="c", subcore_axis_name="s")


@functools.partial(
    pl.kernel,
    out_type=(
        jax.ShapeDtypeStruct((NC, N_PAD, D), jnp.float32),
        jax.ShapeDtypeStruct((NC, N_PAD, DEGW), jnp.float32),
    ),
    mesh=_mesh,
    compiler_params=pltpu.CompilerParams(use_tc_tiling_on_sc=False,
                                         needs_layout_passes=False),
    scratch_types=[
        pltpu.VMEM((GRP, CHUNK), jnp.int32),
        pltpu.VMEM((2 * GRP, HCH), jnp.int32),
        pltpu.VMEM((CHUNK, D), jnp.bfloat16),
        pltpu.VMEM((CHUNK, D), jnp.bfloat16),
        pltpu.VMEM((CHUNK, D), jnp.float32),
        pltpu.VMEM((HCH, DEGW), jnp.float32),
        pltpu.VMEM((CHUNK, DEGW), jnp.float32),
        pltpu.VMEM_SHARED((N_PAD, D), jnp.float32),
        pltpu.VMEM_SHARED((N_PAD, DEGW), jnp.float32),
        pltpu.SemaphoreType.DMA,
        pltpu.SemaphoreType.DMA,
        pltpu.SemaphoreType.DMA,
        pltpu.SemaphoreType.DMA,
        pltpu.SemaphoreType.DMA,
        pltpu.SemaphoreType.DMA,
    ],
)
def _sc_aggregate(xh_hbm, src_hbm, dst_hbm, degc_hbm, agg_out, deg_out,
                  src_v, dst_v, rb0_v, rb1_v, rf_v, ones_v, z8_v,
                  agg_sh, deg_sh, gsem0, gsem1, asem0, asem1, dsem0, dsem1):
    c = lax.axis_index("c")
    s = lax.axis_index("s")
    wid = c * NS + s

    zero16 = jnp.zeros((16,), jnp.float32)

    pltpu.sync_copy(degc_hbm.at[pl.ds(0, HCH)], ones_v)
    pltpu.sync_copy(degc_hbm.at[pl.ds(HCH, CHUNK)], z8_v)

    @pl.loop(0, CHUNK, unroll=4)
    def _fill_rows(i):
        for j in range(D // 16):
            rf_v[i, pl.ds(j * 16, 16)] = zero16

    base = s * ROWS_PER_TILE
    nfull = ROWS_PER_TILE // CHUNK
    for t in range(nfull):
        pltpu.sync_copy(rf_v, agg_sh.at[pl.ds(base + t * CHUNK, CHUNK)])
        pltpu.sync_copy(z8_v, deg_sh.at[pl.ds(base + t * CHUNK, CHUNK)])
    rem = ROWS_PER_TILE % CHUNK
    if rem:
        pltpu.sync_copy(rf_v.at[pl.ds(0, rem)],
                        agg_sh.at[pl.ds(base + nfull * CHUNK, rem)])
        pltpu.sync_copy(z8_v.at[pl.ds(0, rem)],
                        deg_sh.at[pl.ds(base + nfull * CHUNK, rem)])
    plsc.subcore_barrier()

    rbufs = (rb0_v, rb1_v)
    gsems = (gsem0, gsem1)
    asems = (asem0, asem1)
    dsems = (dsem0, dsem1)

    @pl.loop(0, NGRP)
    def _edges(g):
        pltpu.sync_copy(src_hbm.at[wid, pl.ds(g * GRP, GRP)], src_v)
        pltpu.sync_copy(dst_hbm.at[wid, pl.ds(2 * g * GRP, 2 * GRP)], dst_v)
        gd = [None, None]
        sd = [None, None]
        dd = [None, None]
        gd[0] = pltpu.async_copy(xh_hbm.at[src_v.at[0]], rbufs[0], gsems[0])
        gd[1] = pltpu.async_copy(xh_hbm.at[src_v.at[1]], rbufs[1], gsems[1])
        for j in range(GRP):
            b = j & 1
            gd[b].wait()
            rb = rbufs[b]
            for h in range(2):
                if sd[h] is not None:
                    sd[h].wait()

                @pl.loop(HCH * h, HCH * h + HCH, unroll=8)
                def _widen(i):
                    for q in range(D // 32):
                        w = plsc.bitcast(rb[i, pl.ds(32 * q, 32)], jnp.int32)
                        rf_v[i, pl.ds(32 * q, 16)] = plsc.bitcast(
                            w << 16, jnp.float32)
                        rf_v[i, pl.ds(32 * q + 16, 16)] = plsc.bitcast(
                            w & jnp.int32(-65536), jnp.float32)

                k = 2 * j + h
                sd[h] = pltpu.async_copy(rf_v.at[pl.ds(HCH * h, HCH)],
                                         agg_sh.at[dst_v.at[k]],
                                         asems[h], add=True)
                if dd[h] is not None:
                    dd[h].wait()
                dd[h] = pltpu.async_copy(ones_v, deg_sh.at[dst_v.at[k]],
                                         dsems[h], add=True)
            if j + 2 < GRP:
                gd[b] = pltpu.async_copy(xh_hbm.at[src_v.at[j + 2]],
                                         rbufs[b], gsems[b])
        for h in range(2):
            sd[h].wait()
            dd[h].wait()

    plsc.subcore_barrier()

    pltpu.sync_copy(agg_sh.at[pl.ds(base, ROWS_PER_TILE)],
                    agg_out.at[c, pl.ds(base, ROWS_PER_TILE)])
    pltpu.sync_copy(deg_sh.at[pl.ds(base, ROWS_PER_TILE)],
                    deg_out.at[c, pl.ds(base, ROWS_PER_TILE)])


_BLK = 2000


def _tc_body(x_ref, agg_ref, deg_ref, ws_ref, wn_ref, b_ref, o_ref):
    agg = agg_ref[0] + agg_ref[1]
    deg = deg_ref[0, :, 0:1] + deg_ref[1, :, 0:1]
    hn = agg / jnp.maximum(deg, 1.0)
    h = jnp.dot(x_ref[...], ws_ref[...], preferred_element_type=jnp.float32)
    h = h + jnp.dot(hn, wn_ref[...], preferred_element_type=jnp.float32)
    h = h + b_ref[...]
    h = jnp.where(h >= 0.0, h, h * 0.01)
    n2 = jnp.sum(h * h, axis=1, keepdims=True)
    o_ref[...] = h * lax.rsqrt(jnp.maximum(n2, 1e-24))


def _tc_finish(x, agg, deg, W_self, W_neigh, b2):
    grid = (N_NODES // _BLK,)
    return pl.pallas_call(
        _tc_body,
        grid=grid,
        in_specs=[
            pl.BlockSpec((_BLK, D), lambda i: (i, 0)),
            pl.BlockSpec((NC, _BLK, D), lambda i: (0, i, 0)),
            pl.BlockSpec((NC, _BLK, DEGW), lambda i: (0, i, 0)),
            pl.BlockSpec((D, D), lambda i: (0, 0)),
            pl.BlockSpec((D, D), lambda i: (0, 0)),
            pl.BlockSpec((1, D), lambda i: (0, 0)),
        ],
        out_specs=pl.BlockSpec((_BLK, D), lambda i: (i, 0)),
        out_shape=jax.ShapeDtypeStruct((N_NODES, D), jnp.float32),
    )(x, agg, deg, W_self, W_neigh, b2)


def kernel(x, edge_index, W_self, W_neigh, b):
    src = edge_index[0]
    dst = edge_index[1]
    xh = x.astype(jnp.bfloat16)
    real_pw = N_EDGES // NW
    pad_pw = EPW - real_pw
    pad_src = jnp.zeros((NW, pad_pw), jnp.int32)
    pad_dst = jnp.broadcast_to(
        N_NODES + (jnp.arange(pad_pw, dtype=jnp.int32) % (N_PAD - N_NODES)),
        (NW, pad_pw))
    src_p = jnp.concatenate([src.reshape(NW, real_pw), pad_src], axis=1)
    dst_p = jnp.concatenate([dst.reshape(NW, real_pw), pad_dst], axis=1)
    src_p = src_p.reshape(NW, NCHUNK, CHUNK)
    dst_p = dst_p.reshape(NW, 2 * NCHUNK, HCH)
    agg, deg = _sc_aggregate(xh, src_p, dst_p, jnp.asarray(_DEGC))
    wn = W_neigh[jnp.asarray(_PERM), :]
    return _tc_finish(x, agg, deg, W_self, wn, b.reshape(1, D))


# speedup vs baseline: 1.1087x; 1.0003x over previous
import functools

import jax
import jax.numpy as jnp
import numpy as np
from jax import lax
from jax.experimental import pallas as pl
from jax.experimental.pallas import tpu as pltpu
from jax.experimental.pallas import tpu_sc as plsc

N_NODES = 10000
N_EDGES = 320000
D = 128

NC = 2
NS = 16
NW = NC * NS
CHUNK = 128
GRP = 8
NGRP = 10
NCHUNK = GRP * NGRP
EPW = NCHUNK * CHUNK
N_PAD = 10112
ROWS_PER_TILE = N_PAD // NS
DEGW = 8
NSPL = 2
HCH = CHUNK // NSPL

_PERM = np.empty((D,), dtype=np.int32)
for _g in range(D // 32):
    for _k in range(16):
        _PERM[32 * _g + _k] = 32 * _g + 2 * _k
        _PERM[32 * _g + 16 + _k] = 32 * _g + 2 * _k + 1

_DEGC = np.zeros((HCH + CHUNK, DEGW), dtype=np.float32)
_DEGC[:HCH, 0] = 1.0

_mesh = plsc.VectorSubcoreMesh(core_axis_name="c", subcore_axis_name="s")


@functools.partial(
    pl.kernel,
    out_type=(
        jax.ShapeDtypeStruct((NC, N_PAD, D), jnp.float32),
        jax.ShapeDtypeStruct((NC, N_PAD, DEGW), jnp.float32),
    ),
    mesh=_mesh,
    compiler_params=pltpu.CompilerParams(use_tc_tiling_on_sc=False,
                                         needs_layout_passes=False),
    scratch_types=[
        pltpu.VMEM((GRP, CHUNK), jnp.int32),
        pltpu.VMEM((NSPL * GRP, HCH), jnp.int32),
        pltpu.VMEM((CHUNK, D), jnp.bfloat16),
        pltpu.VMEM((CHUNK, D), jnp.bfloat16),
        pltpu.VMEM((CHUNK, D), jnp.float32),
        pltpu.VMEM((HCH, DEGW), jnp.float32),
        pltpu.VMEM((CHUNK, DEGW), jnp.float32),
        pltpu.VMEM_SHARED((N_PAD, D), jnp.float32),
        pltpu.VMEM_SHARED((N_PAD, DEGW), jnp.float32),
        pltpu.SemaphoreType.DMA,
        pltpu.SemaphoreType.DMA,
        pltpu.SemaphoreType.DMA,
        pltpu.SemaphoreType.DMA,
        pltpu.SemaphoreType.DMA,
        pltpu.SemaphoreType.DMA,
        pltpu.SemaphoreType.DMA,
        pltpu.SemaphoreType.DMA,
        pltpu.SemaphoreType.DMA,
        pltpu.SemaphoreType.DMA,
    ],
)
def _sc_aggregate(xh_hbm, src_hbm, dst_hbm, degc_hbm, agg_out, deg_out,
                  src_v, dst_v, rb0_v, rb1_v, rf_v, ones_v, z8_v,
                  agg_sh, deg_sh, gsem0, gsem1,
                  asem0, asem1, asem2, asem3, dsem0, dsem1, dsem2, dsem3):
    c = lax.axis_index("c")
    s = lax.axis_index("s")
    wid = c * NS + s

    zero16 = jnp.zeros((16,), jnp.float32)

    pltpu.sync_copy(degc_hbm.at[pl.ds(0, HCH)], ones_v)
    pltpu.sync_copy(degc_hbm.at[pl.ds(HCH, CHUNK)], z8_v)

    @pl.loop(0, CHUNK, unroll=4)
    def _fill_rows(i):
        for j in range(D // 16):
            rf_v[i, pl.ds(j * 16, 16)] = zero16

    base = s * ROWS_PER_TILE
    nfull = ROWS_PER_TILE // CHUNK
    for t in range(nfull):
        pltpu.sync_copy(rf_v, agg_sh.at[pl.ds(base + t * CHUNK, CHUNK)])
        pltpu.sync_copy(z8_v, deg_sh.at[pl.ds(base + t * CHUNK, CHUNK)])
    rem = ROWS_PER_TILE % CHUNK
    if rem:
        pltpu.sync_copy(rf_v.at[pl.ds(0, rem)],
                        agg_sh.at[pl.ds(base + nfull * CHUNK, rem)])
        pltpu.sync_copy(z8_v.at[pl.ds(0, rem)],
                        deg_sh.at[pl.ds(base + nfull * CHUNK, rem)])
    plsc.subcore_barrier()

    rbufs = (rb0_v, rb1_v)
    gsems = (gsem0, gsem1)
    asems = (asem0, asem1, asem2, asem3)
    dsems = (dsem0, dsem1, dsem2, dsem3)

    @pl.loop(0, NGRP)
    def _edges(g):
        pltpu.sync_copy(src_hbm.at[wid, pl.ds(g * GRP, GRP)], src_v)
        pltpu.sync_copy(dst_hbm.at[wid, pl.ds(NSPL * g * GRP, NSPL * GRP)],
                        dst_v)
        gd = [None, None]
        sd = [None] * NSPL
        dd = [None] * NSPL
        gd[0] = pltpu.async_copy(xh_hbm.at[src_v.at[0]], rbufs[0], gsems[0])
        gd[1] = pltpu.async_copy(xh_hbm.at[src_v.at[1]], rbufs[1], gsems[1])
        for j in range(GRP):
            b = j & 1
            gd[b].wait()
            rb = rbufs[b]
            for h in range(NSPL):
                if sd[h] is not None:
                    sd[h].wait()

                @pl.loop(HCH * h, HCH * h + HCH, unroll=8)
                def _widen(i):
                    for q in range(D // 32):
                        w = plsc.bitcast(rb[i, pl.ds(32 * q, 32)], jnp.int32)
                        rf_v[i, pl.ds(32 * q, 16)] = plsc.bitcast(
                            w << 16, jnp.float32)
                        rf_v[i, pl.ds(32 * q + 16, 16)] = plsc.bitcast(
                            w & jnp.int32(-65536), jnp.float32)

                k = NSPL * j + h
                sd[h] = pltpu.async_copy(rf_v.at[pl.ds(HCH * h, HCH)],
                                         agg_sh.at[dst_v.at[k]],
                                         asems[h], add=True)
                if dd[h] is not None:
                    dd[h].wait()
                dd[h] = pltpu.async_copy(ones_v, deg_sh.at[dst_v.at[k]],
                                         dsems[h], add=True)
            if j + 2 < GRP:
                gd[b] = pltpu.async_copy(xh_hbm.at[src_v.at[j + 2]],
                                         rbufs[b], gsems[b])
        for h in range(NSPL):
            sd[h].wait()
            dd[h].wait()

    plsc.subcore_barrier()

    pltpu.sync_copy(agg_sh.at[pl.ds(base, ROWS_PER_TILE)],
                    agg_out.at[c, pl.ds(base, ROWS_PER_TILE)])
    pltpu.sync_copy(deg_sh.at[pl.ds(base, ROWS_PER_TILE)],
                    deg_out.at[c, pl.ds(base, ROWS_PER_TILE)])


_BLK = 2000


def _tc_body(x_ref, agg_ref, deg_ref, ws_ref, wn_ref, b_ref, o_ref):
    agg = agg_ref[0] + agg_ref[1]
    deg = deg_ref[0, :, 0:1] + deg_ref[1, :, 0:1]
    hn = agg / jnp.maximum(deg, 1.0)
    h = jnp.dot(x_ref[...], ws_ref[...], preferred_element_type=jnp.float32)
    h = h + jnp.dot(hn, wn_ref[...], preferred_element_type=jnp.float32)
    h = h + b_ref[...]
    h = jnp.where(h >= 0.0, h, h * 0.01)
    n2 = jnp.sum(h * h, axis=1, keepdims=True)
    o_ref[...] = h * lax.rsqrt(jnp.maximum(n2, 1e-24))


def _tc_finish(x, agg, deg, W_self, W_neigh, b2):
    grid = (N_NODES // _BLK,)
    return pl.pallas_call(
        _tc_body,
        grid=grid,
        in_specs=[
            pl.BlockSpec((_BLK, D), lambda i: (i, 0)),
            pl.BlockSpec((NC, _BLK, D), lambda i: (0, i, 0)),
            pl.BlockSpec((NC, _BLK, DEGW), lambda i: (0, i, 0)),
            pl.BlockSpec((D, D), lambda i: (0, 0)),
            pl.BlockSpec((D, D), lambda i: (0, 0)),
            pl.BlockSpec((1, D), lambda i: (0, 0)),
        ],
        out_specs=pl.BlockSpec((_BLK, D), lambda i: (i, 0)),
        out_shape=jax.ShapeDtypeStruct((N_NODES, D), jnp.float32),
    )(x, agg, deg, W_self, W_neigh, b2)


def kernel(x, edge_index, W_self, W_neigh, b):
    src = edge_index[0]
    dst = edge_index[1]
    xh = x.astype(jnp.bfloat16)
    real_pw = N_EDGES // NW
    pad_pw = EPW - real_pw
    pad_src = jnp.zeros((NW, pad_pw), jnp.int32)
    pad_dst = jnp.broadcast_to(
        N_NODES + (jnp.arange(pad_pw, dtype=jnp.int32) % (N_PAD - N_NODES)),
        (NW, pad_pw))
    src_p = jnp.concatenate([src.reshape(NW, real_pw), pad_src], axis=1)
    dst_p = jnp.concatenate([dst.reshape(NW, real_pw), pad_dst], axis=1)
    src_p = src_p.reshape(NW, NCHUNK, CHUNK)
    dst_p = dst_p.reshape(NW, NSPL * NCHUNK, HCH)
    agg, deg = _sc_aggregate(xh, src_p, dst_p, jnp.asarray(_DEGC))
    wn = W_neigh[jnp.asarray(_PERM), :]
    return _tc_finish(x, agg, deg, W_self, wn, b.reshape(1, D))
